# Initial kernel scaffold; baseline (speedup 1.0000x reference)
import jax
import jax.numpy as jnp
from jax.experimental import pallas as pl


def kernel(x, W1, b1, W2, b2):
    raise NotImplementedError("write your pallas kernel here")


# speedup vs baseline: 8.6976x; 8.6976x over previous
import functools

import jax
import jax.numpy as jnp
from jax import lax
from jax.experimental import pallas as pl
from jax.experimental.pallas import tpu as pltpu

_B, _C, _N, _K, _OUT = 16, 64, 1024, 20, 64
_RB = 256


def _edgeconv_body(xb_ref, xr_ref, w1d_ref, w1b_ref, b1_ref, w2_ref, b2_ref,
                   out_ref, cur_ref):
    xb = xb_ref[0]
    xr = xr_ref[0]
    sqb = jnp.sum(xb * xb, axis=1)
    sqr = jnp.sum(xr * xr, axis=1)
    dot = lax.dot_general(xr, xb, (((1,), (1,)), ((), ())),
                          preferred_element_type=jnp.float32)
    cur_ref[...] = sqr[:, None] - 2.0 * dot + sqb[None, :]

    v = jnp.dot(xb, w1b_ref[...], preferred_element_type=jnp.float32)
    u = jnp.dot(xr, w1d_ref[...], preferred_element_type=jnp.float32)
    ub = u + b1_ref[...][None, :]
    w2 = w2_ref[...]
    b2 = b2_ref[...][None, :]

    iota = lax.broadcasted_iota(jnp.int32, (_RB, _N), 1)

    def body(_, acc):
        cur = cur_ref[...]
        m = jnp.min(cur, axis=1, keepdims=True)
        ismin = cur == m
        amin = jnp.min(jnp.where(ismin, iota, _N), axis=1, keepdims=True)
        onehot = iota == amin
        vj = jnp.dot(onehot.astype(jnp.float32), v,
                     preferred_element_type=jnp.float32)
        e = jnp.maximum(ub + vj, 0.0)
        o = jnp.maximum(jnp.dot(e, w2, preferred_element_type=jnp.float32) + b2,
                        0.0)
        cur_ref[...] = jnp.where(onehot, jnp.inf, cur)
        return jnp.maximum(acc, o)

    acc = lax.fori_loop(0, _K, body,
                        jnp.full((_RB, _OUT), -jnp.inf, jnp.float32))
    out_ref[0] = acc


@functools.partial(jax.jit, static_argnames=("interpret",))
def kernel(x, W1, b1, W2, b2, interpret=False):
    xf = jnp.transpose(x[..., 0], (0, 2, 1))
    w1a, w1b = W1[:_C], W1[_C:]
    w1d = w1a - w1b

    grid = (_B, _N // _RB)
    out = pl.pallas_call(
        _edgeconv_body,
        grid=grid,
        in_specs=[
            pl.BlockSpec((1, _N, _C), lambda b, r: (b, 0, 0)),
            pl.BlockSpec((1, _RB, _C), lambda b, r: (b, r, 0)),
            pl.BlockSpec((_C, _OUT), lambda b, r: (0, 0)),
            pl.BlockSpec((_C, _OUT), lambda b, r: (0, 0)),
            pl.BlockSpec((_OUT,), lambda b, r: (0,)),
            pl.BlockSpec((_OUT, _OUT), lambda b, r: (0, 0)),
            pl.BlockSpec((_OUT,), lambda b, r: (0,)),
        ],
        out_specs=pl.BlockSpec((1, _RB, _OUT), lambda b, r: (b, r, 0)),
        out_shape=jax.ShapeDtypeStruct((_B, _N, _OUT), jnp.float32),
        scratch_shapes=[pltpu.VMEM((_RB, _N), jnp.float32)],
        interpret=interpret,
    )(xf, xf, w1d, w1b, b1, W2, b2)
    return jnp.transpose(out, (0, 2, 1))[..., None]
